# baseline (speedup 1.0000x reference)
import jax
import jax.numpy as jnp
from jax import lax
from jax.experimental import pallas as pl
from jax.experimental.pallas import tpu as pltpu

N, K, E, O, I = 2048, 2, 64, 128, 128
M = O * I

TN = 128
NB = 4
GRID = N // TN


def _mix_kernel(wp_ref, wi_ref, bp_ref, bi_ref, bank_ref, bbank_ref,
                out_hbm, bout_ref, bufs, sems):
    i = pl.program_id(0)
    slot = lax.rem(i, NB)
    t0 = i * TN

    wp = wp_ref[...]
    wi = wi_ref[...]
    iota = lax.broadcasted_iota(jnp.int32, (TN, E), 1)
    s = (wp[:, 0:1] * (wi[:, 0:1] == iota).astype(jnp.float32)
         + wp[:, 1:2] * (wi[:, 1:2] == iota).astype(jnp.float32))
    bank = bank_ref[...].reshape(E, M)
    res = jnp.dot(s, bank, preferred_element_type=jnp.float32)

    @pl.when(i >= NB)
    def _():
        pltpu.make_async_copy(
            bufs.at[slot], out_hbm.at[pl.ds(t0 - NB * TN, TN)],
            sems.at[slot]).wait()

    bufs[slot] = res.reshape(TN, O, I)
    pltpu.make_async_copy(bufs.at[slot], out_hbm.at[pl.ds(t0, TN)],
                          sems.at[slot]).start()

    bp = bp_ref[...]
    bi = bi_ref[...]
    sb = (bp[:, 0:1] * (bi[:, 0:1] == iota).astype(jnp.float32)
          + bp[:, 1:2] * (bi[:, 1:2] == iota).astype(jnp.float32))
    bout_ref[...] = jnp.dot(sb, bbank_ref[...],
                            preferred_element_type=jnp.float32)

    @pl.when(i == GRID - 1)
    def _():
        for k in range(NB):
            step = GRID - NB + k
            pltpu.make_async_copy(
                bufs.at[step % NB], out_hbm.at[pl.ds(step * TN, TN)],
                sems.at[step % NB]).wait()


def kernel(weight_probs, weight_indices, bias_probs, bias_indices,
           weight_bank, bias_bank):
    wi = weight_indices.astype(jnp.int32)
    bi = bias_indices.astype(jnp.int32)

    out, bout = pl.pallas_call(
        _mix_kernel,
        grid=(GRID,),
        in_specs=[
            pl.BlockSpec((TN, K), lambda i: (i, 0)),
            pl.BlockSpec((TN, K), lambda i: (i, 0)),
            pl.BlockSpec((TN, K), lambda i: (i, 0)),
            pl.BlockSpec((TN, K), lambda i: (i, 0)),
            pl.BlockSpec((E, O, I), lambda i: (0, 0, 0)),
            pl.BlockSpec((E, O), lambda i: (0, 0)),
        ],
        out_specs=[
            pl.BlockSpec(memory_space=pl.ANY),
            pl.BlockSpec((TN, O), lambda i: (i, 0)),
        ],
        out_shape=[
            jax.ShapeDtypeStruct((N, O, I), jnp.float32),
            jax.ShapeDtypeStruct((N, O), jnp.float32),
        ],
        scratch_shapes=[
            pltpu.VMEM((NB, TN, O, I), jnp.float32),
            pltpu.SemaphoreType.DMA((NB,)),
        ],
    )(weight_probs, wi, bias_probs, bi, weight_bank, bias_bank)

    return out, bout
